# baseline (speedup 1.0000x reference)
import functools

import jax
import jax.numpy as jnp
from jax import lax
from jax.experimental import pallas as pl
from jax.experimental.pallas import tpu as pltpu
from jax.experimental.pallas import tpu_sc as plsc

_BATCH = 16384
_F = 26
_BINS = 100000
_D = 64
_B = _BATCH * _F
_NW = 32
_BPW = _B // _NW
_C = 128
_T = _BPW // _C
_ROWS = _BPW // 128
_NBUF = 4


def _sc_gather(t3, xf):
    mesh = plsc.VectorSubcoreMesh(core_axis_name="c", subcore_axis_name="s")

    scratch = [
        pltpu.VMEM((_ROWS, 128), jnp.int32),
        pltpu.VMEM((_ROWS, 128), jnp.int32),
    ]
    scratch += [pltpu.VMEM((_C, 128), jnp.float32) for _ in range(_NBUF)]
    scratch += [pltpu.SemaphoreType.DMA for _ in range(2 * _NBUF)]

    @functools.partial(
        pl.kernel,
        out_type=jax.ShapeDtypeStruct((_B, 128), jnp.float32),
        mesh=mesh,
        scratch_types=scratch,
        compiler_params=pltpu.CompilerParams(
            use_tc_tiling_on_sc=True, needs_layout_passes=False),
    )
    def body(t3_hbm, xf_hbm, out_hbm, xv, pv, *rest):
        bufs = rest[:_NBUF]
        gsem = rest[_NBUF:2 * _NBUF]
        psem = rest[2 * _NBUF:]

        wid = lax.axis_index("s") * 2 + lax.axis_index("c")
        base = wid * _BPW
        rbase = pl.multiple_of(wid * _ROWS, 8)
        lanes = lax.iota(jnp.int32, 16)

        pltpu.sync_copy(xf_hbm.at[pl.ds(rbase, _ROWS), :], xv)

        @pl.loop(0, _BPW // 16)
        def _(i):
            r = i // 8
            cc = i - r * 8
            pos = i * 16 + lanes
            f = lax.rem(pos, _F)
            flat = xv[r, pl.ds(cc * 16, 16)] + f * _BINS
            pv[r, pl.ds(cc * 16, 16)] = lax.shift_right_logical(flat, 1)

        def start_gather(j, b):
            pltpu.async_copy(t3_hbm.at[pv.at[j]], bufs[b], gsem[b])

        for b in range(_NBUF):
            start_gather(b, b)

        def drain(j, b, with_next):
            pltpu.make_async_copy(
                t3_hbm.at[pv.at[j]], bufs[b], gsem[b]).wait()

            jv = jnp.zeros((16,), jnp.int32) + j

            @pl.loop(0, _C)
            def _(i):
                iv = jnp.zeros((16,), jnp.int32) + i
                hv = jnp.bitwise_and(
                    plsc.load_gather(xv.at[:, :], [jv, iv]), 1)
                off = hv * _D
                for q in range(4):
                    vals = plsc.load_gather(
                        bufs[b].at[:, :], [iv, off + q * 16 + lanes])
                    bufs[b][i, pl.ds(q * 16, 16)] = vals

            pltpu.async_copy(
                bufs[b],
                out_hbm.at[pl.ds(pl.multiple_of(base + j * _C, 128), _C), :],
                psem[b]).wait()
            if with_next:
                start_gather(j + _NBUF, b)

        @pl.loop(0, _T - _NBUF, step=_NBUF)
        def _(g):
            for b in range(_NBUF):
                drain(g + b, b, True)

        for b in range(_NBUF):
            drain(_T - _NBUF + b, b, False)

    return body(t3, xf)


def kernel(x, tables):
    t3 = tables.reshape(_F * _BINS // 2, 128)
    xf = x.reshape(_B // 128, 128)
    outp = _sc_gather(t3, xf)
    return outp[:, :_D].reshape(_BATCH, _F * _D)
